# baseline (speedup 1.0000x reference)
import functools

import numpy as np
import jax
import jax.numpy as jnp
from jax import lax
from jax.experimental import pallas as pl
from jax.experimental.pallas import tpu as pltpu
from jax.experimental.pallas import tpu_sc as plsc

_EPS = 1e-5
_MAX_LEN = 512


def _positional_encoding(max_len, d):
    pos = np.arange(max_len, dtype=np.float32)[:, None]
    div = np.exp(np.arange(0, d, 2, dtype=np.float32) * (-np.log(10000.0) / d))
    pe = np.zeros((max_len, d), dtype=np.float32)
    pe[:, 0::2] = np.sin(pos * div)
    pe[:, 1::2] = np.cos(pos * div)
    return pe


@functools.lru_cache(maxsize=None)
def _make_sc_gather(V, D, BT):
    info = plsc.get_sparse_core_info()
    NC, NS = info.num_cores, info.num_subcores
    NW = NC * NS
    assert BT % NW == 0
    b_per_w = BT // NW
    C = 512
    assert b_per_w % C == 0
    nch = b_per_w // C
    mesh = plsc.VectorSubcoreMesh(core_axis_name="c", subcore_axis_name="s")

    @functools.partial(
        pl.kernel,
        mesh=mesh,
        compiler_params=pltpu.CompilerParams(use_tc_tiling_on_sc=False),
        out_type=jax.ShapeDtypeStruct((BT, D), jnp.float32),
        scratch_types=[
            pltpu.VMEM((b_per_w,), jnp.int32),
            pltpu.VMEM((C, D), jnp.float32),
            pltpu.VMEM((C, D), jnp.float32),
            pltpu.SemaphoreType.DMA,
            pltpu.SemaphoreType.DMA,
            pltpu.SemaphoreType.DMA,
            pltpu.SemaphoreType.DMA,
        ],
    )
    def gather(table_hbm, idx_hbm, out_hbm, idx_v, buf0, buf1, gs0, gs1, os0, os1):
        wid = lax.axis_index("s") * NC + lax.axis_index("c")
        base = wid * b_per_w
        pltpu.sync_copy(idx_hbm.at[pl.ds(base, b_per_w)], idx_v)
        bufs = (buf0, buf1)
        gsems = (gs0, gs1)
        osems = (os0, os1)
        gcp = [None, None]
        ocp = [None, None]
        gcp[0] = pltpu.async_copy(table_hbm.at[idx_v.at[pl.ds(0, C)]], bufs[0], gsems[0])
        for c in range(nch):
            i = c & 1
            nxt = c + 1
            if nxt < nch:
                j = nxt & 1
                if ocp[j] is not None:
                    ocp[j].wait()
                gcp[j] = pltpu.async_copy(
                    table_hbm.at[idx_v.at[pl.ds(nxt * C, C)]], bufs[j], gsems[j]
                )
            gcp[i].wait()
            ocp[i] = pltpu.async_copy(bufs[i], out_hbm.at[pl.ds(base + c * C, C)], osems[i])
        for o in ocp:
            if o is not None:
                o.wait()

    return gather


@functools.lru_cache(maxsize=None)
def _make_tc_dense(BT, R, D, M):
    def body(x_ref, pe_ref, w_ref, b_ref, g_ref, be_ref, o_ref):
        x = x_ref[...] + pe_ref[...]
        y = lax.dot_general(
            x, w_ref[...], (((1,), (1,)), ((), ())), preferred_element_type=jnp.float32
        )
        y = y + b_ref[...]
        mean = jnp.mean(y, axis=1, keepdims=True)
        d = y - mean
        var = jnp.mean(d * d, axis=1, keepdims=True)
        o_ref[...] = d * lax.rsqrt(var + _EPS) * g_ref[...] + be_ref[...]

    return pl.pallas_call(
        body,
        grid=(BT // R,),
        in_specs=[
            pl.BlockSpec((R, D), lambda i: (i, 0)),
            pl.BlockSpec((R, D), lambda i: (0, 0)),
            pl.BlockSpec((M, D), lambda i: (0, 0)),
            pl.BlockSpec((1, M), lambda i: (0, 0)),
            pl.BlockSpec((1, M), lambda i: (0, 0)),
            pl.BlockSpec((1, M), lambda i: (0, 0)),
        ],
        out_specs=pl.BlockSpec((R, M), lambda i: (i, 0)),
        out_shape=jax.ShapeDtypeStruct((BT, M), jnp.float32),
    )


def kernel(sequence, table, W, b, gamma, beta):
    B, L = sequence.shape
    V, D = table.shape
    M = W.shape[0]
    BT = B * L
    idx = sequence.reshape(BT)
    tok = _make_sc_gather(V, D, BT)(table, idx)
    pe = jnp.asarray(_positional_encoding(_MAX_LEN, D)[:L])
    out = _make_tc_dense(BT, L, D, M)(
        tok,
        pe,
        W,
        b.reshape(1, M),
        gamma.reshape(1, M),
        beta.reshape(1, M),
    )
    return out.reshape(B, L, M)


# speedup vs baseline: 1.7465x; 1.7465x over previous
import functools

import numpy as np
import jax
import jax.numpy as jnp
from jax import lax
from jax.experimental import pallas as pl
from jax.experimental.pallas import tpu as pltpu
from jax.experimental.pallas import tpu_sc as plsc

_EPS = 1e-5
_MAX_LEN = 512


def _positional_encoding(max_len, d):
    pos = np.arange(max_len, dtype=np.float32)[:, None]
    div = np.exp(np.arange(0, d, 2, dtype=np.float32) * (-np.log(10000.0) / d))
    pe = np.zeros((max_len, d), dtype=np.float32)
    pe[:, 0::2] = np.sin(pos * div)
    pe[:, 1::2] = np.cos(pos * div)
    return pe


@functools.lru_cache(maxsize=None)
def _make_sc_gather(V, D, BT, W_OUT):
    info = plsc.get_sparse_core_info()
    NC, NS = info.num_cores, info.num_subcores
    NW = NC * NS
    assert BT % NW == 0
    b_per_w = BT // NW
    C = 512
    assert b_per_w % C == 0
    nch = b_per_w // C
    mesh = plsc.VectorSubcoreMesh(core_axis_name="c", subcore_axis_name="s")

    @functools.partial(
        pl.kernel,
        mesh=mesh,
        compiler_params=pltpu.CompilerParams(use_tc_tiling_on_sc=False),
        out_type=jax.ShapeDtypeStruct((BT, W_OUT), jnp.float32),
        scratch_types=[
            pltpu.VMEM((b_per_w,), jnp.int32),
            pltpu.VMEM((C, D), jnp.float32),
            pltpu.VMEM((C, D), jnp.float32),
            pltpu.SemaphoreType.DMA,
            pltpu.SemaphoreType.DMA,
            pltpu.SemaphoreType.DMA,
            pltpu.SemaphoreType.DMA,
        ],
    )
    def gather(table_hbm, idx_hbm, out_hbm, idx_v, buf0, buf1, gs0, gs1, os0, os1):
        wid = lax.axis_index("s") * NC + lax.axis_index("c")
        base = wid * b_per_w
        pltpu.sync_copy(idx_hbm.at[pl.ds(base, b_per_w)], idx_v)
        bufs = (buf0, buf1)
        gsems = (gs0, gs1)
        osems = (os0, os1)
        gcp = [None, None]
        ocp = [None, None]
        gcp[0] = pltpu.async_copy(table_hbm.at[idx_v.at[pl.ds(0, C)]], bufs[0], gsems[0])
        for c in range(nch):
            i = c & 1
            nxt = c + 1
            if nxt < nch:
                j = nxt & 1
                if ocp[j] is not None:
                    ocp[j].wait()
                gcp[j] = pltpu.async_copy(
                    table_hbm.at[idx_v.at[pl.ds(nxt * C, C)]], bufs[j], gsems[j]
                )
            gcp[i].wait()
            ocp[i] = pltpu.async_copy(
                bufs[i], out_hbm.at[pl.ds(base + c * C, C), pl.ds(0, D)], osems[i]
            )
        for o in ocp:
            if o is not None:
                o.wait()

    return gather


@functools.lru_cache(maxsize=None)
def _make_tc_dense(BT, R, L, D, M, W_IN):
    S = R // L

    def body(x_ref, pe_ref, w_ref, b_ref, g_ref, be_ref, o_ref):
        x = x_ref[...]
        col = lax.broadcasted_iota(jnp.int32, (R, W_IN), 1)
        x = jnp.where(col < D, x, 0.0)
        pe = pe_ref[...]
        if S > 1:
            x = x.reshape(S, L, W_IN) + pe[None, :, :]
            x = x.reshape(R, W_IN)
        else:
            x = x + pe
        y = lax.dot_general(
            x, w_ref[...], (((1,), (1,)), ((), ())), preferred_element_type=jnp.float32
        )
        y = y + b_ref[...]
        mean = jnp.mean(y, axis=1, keepdims=True)
        d = y - mean
        var = jnp.mean(d * d, axis=1, keepdims=True)
        o_ref[...] = d * lax.rsqrt(var + _EPS) * g_ref[...] + be_ref[...]

    return pl.pallas_call(
        body,
        grid=(BT // R,),
        in_specs=[
            pl.BlockSpec((R, W_IN), lambda i: (i, 0)),
            pl.BlockSpec((L, W_IN), lambda i: (0, 0)),
            pl.BlockSpec((M, W_IN), lambda i: (0, 0)),
            pl.BlockSpec((1, M), lambda i: (0, 0)),
            pl.BlockSpec((1, M), lambda i: (0, 0)),
            pl.BlockSpec((1, M), lambda i: (0, 0)),
        ],
        out_specs=pl.BlockSpec((R, M), lambda i: (i, 0)),
        out_shape=jax.ShapeDtypeStruct((BT, M), jnp.float32),
    )


def kernel(sequence, table, W, b, gamma, beta):
    B, L = sequence.shape
    V, D = table.shape
    M = W.shape[0]
    BT = B * L
    idx = sequence.reshape(BT)
    W_IN = 2 * D
    tok = _make_sc_gather(V, D, BT, W_IN)(table, idx)
    pe_np = np.zeros((L, W_IN), dtype=np.float32)
    pe_np[:, :D] = _positional_encoding(_MAX_LEN, D)[:L]
    pe = jnp.asarray(pe_np)
    W_pad = jnp.pad(W, ((0, 0), (0, W_IN - D)))
    R = 4096
    out = _make_tc_dense(BT, R, L, D, M, W_IN)(
        tok,
        pe,
        W_pad,
        b.reshape(1, M),
        gamma.reshape(1, M),
        beta.reshape(1, M),
    )
    return out.reshape(B, L, M)
